# Initial kernel scaffold; baseline (speedup 1.0000x reference)
import functools

import jax
import jax.numpy as jnp
from jax import lax
from jax.experimental import pallas as pl
from jax.experimental.pallas import tpu as pltpu
from jax.experimental.pallas import tpu_sc as plsc

N = 10000
E = 320000
F_IN = 128
F_HID = 128
F_OUT = 64

NC = 2
NS = 16
NW = NC * NS
CHUNK = 128
CH_PER_W = 80
E_PAD = NW * CH_PER_W * CHUNK
N_ACC = 10240
ROWS_PER_TILE = N_ACC // NS
BR = 1000


def _mesh():
    return plsc.VectorSubcoreMesh(core_axis_name="c", subcore_axis_name="s")


def _deg_body(srcm, dstm, ones_hbm, zeros_hbm, out_o, out_i,
              srcv, dstv, onesb, acc_o, acc_i, gsem):
    cid = lax.axis_index("c")
    sid = lax.axis_index("s")
    wid = sid * NC + cid
    r0 = sid * ROWS_PER_TILE
    pltpu.sync_copy(zeros_hbm, acc_o.at[pl.ds(r0, ROWS_PER_TILE)])
    pltpu.sync_copy(zeros_hbm, acc_i.at[pl.ds(r0, ROWS_PER_TILE)])
    pltpu.sync_copy(ones_hbm, onesb)
    pltpu.sync_copy(srcm.at[pl.ds(wid * CH_PER_W, CH_PER_W)], srcv)
    pltpu.sync_copy(dstm.at[pl.ds(wid * CH_PER_W, CH_PER_W)], dstv)
    plsc.subcore_barrier()

    def body(j, carry):
        pltpu.sync_copy(onesb, acc_o.at[srcv.at[j]], add=True)
        pltpu.sync_copy(onesb, acc_i.at[dstv.at[j]], add=True)
        return carry

    lax.fori_loop(0, CH_PER_W, body, 0)
    plsc.subcore_barrier()
    pltpu.sync_copy(acc_o.at[pl.ds(r0, ROWS_PER_TILE)],
                    out_o.at[cid, pl.ds(r0, ROWS_PER_TILE)])
    pltpu.sync_copy(acc_i.at[pl.ds(r0, ROWS_PER_TILE)],
                    out_i.at[cid, pl.ds(r0, ROWS_PER_TILE)])


def _make_deg():
    return functools.partial(
        pl.kernel,
        mesh=_mesh(),
        out_type=[
            jax.ShapeDtypeStruct((NC, N_ACC, 16), jnp.float32),
            jax.ShapeDtypeStruct((NC, N_ACC, 16), jnp.float32),
        ],
        scratch_types=[
            pltpu.VMEM((CH_PER_W, CHUNK), jnp.int32),
            pltpu.VMEM((CH_PER_W, CHUNK), jnp.int32),
            pltpu.VMEM((CHUNK, 16), jnp.float32),
            pltpu.VMEM_SHARED((N_ACC, 16), jnp.float32),
            pltpu.VMEM_SHARED((N_ACC, 16), jnp.float32),
            pltpu.SemaphoreType.DMA,
        ],
    )(_deg_body)


def _agg_body(y_hbm, srcm, dstm, zeros_hbm, out_hbm,
              srcv, dstv, rowb, acc, gsem):
    cid = lax.axis_index("c")
    sid = lax.axis_index("s")
    wid = sid * NC + cid
    r0 = sid * ROWS_PER_TILE
    pltpu.sync_copy(zeros_hbm, acc.at[pl.ds(r0, ROWS_PER_TILE)])
    pltpu.sync_copy(srcm.at[pl.ds(wid * CH_PER_W, CH_PER_W)], srcv)
    pltpu.sync_copy(dstm.at[pl.ds(wid * CH_PER_W, CH_PER_W)], dstv)
    plsc.subcore_barrier()

    def body(j, carry):
        pltpu.async_copy(y_hbm.at[srcv.at[j]], rowb, gsem).wait()
        pltpu.sync_copy(rowb, acc.at[dstv.at[j]], add=True)
        return carry

    lax.fori_loop(0, CH_PER_W, body, 0)
    plsc.subcore_barrier()
    pltpu.sync_copy(acc.at[pl.ds(r0, ROWS_PER_TILE)],
                    out_hbm.at[cid, pl.ds(r0, ROWS_PER_TILE)])


def _make_agg(C):
    return functools.partial(
        pl.kernel,
        mesh=_mesh(),
        out_type=jax.ShapeDtypeStruct((NC, N_ACC, C), jnp.float32),
        scratch_types=[
            pltpu.VMEM((CH_PER_W, CHUNK), jnp.int32),
            pltpu.VMEM((CH_PER_W, CHUNK), jnp.int32),
            pltpu.VMEM((CHUNK, C), jnp.float32),
            pltpu.VMEM_SHARED((N_ACC, C), jnp.float32),
            pltpu.SemaphoreType.DMA,
        ],
    )(_agg_body)


def _stage0_body(x_ref, w_ref, pdo_ref, o_ref):
    deg = pdo_ref[0] + pdo_ref[1]
    ns = lax.rsqrt(jnp.maximum(deg, 1.0))[:, 0:1]
    o_ref[...] = jnp.dot(x_ref[...] * ns, w_ref[...],
                         preferred_element_type=jnp.float32)


def _stage_mid_body(p_ref, pdi_ref, pdo_ref, b_ref, w_ref, o_ref):
    s = p_ref[0] + p_ref[1]
    nd = lax.rsqrt(jnp.maximum(pdi_ref[0] + pdi_ref[1], 1.0))[:, 0:1]
    h = jnp.maximum(s * nd + b_ref[...], 0.0)
    ns = lax.rsqrt(jnp.maximum(pdo_ref[0] + pdo_ref[1], 1.0))[:, 0:1]
    o_ref[...] = jnp.dot(h * ns, w_ref[...],
                         preferred_element_type=jnp.float32)


def _stage_fin_body(p_ref, pdi_ref, b_ref, o_ref):
    s = p_ref[0] + p_ref[1]
    nd = lax.rsqrt(jnp.maximum(pdi_ref[0] + pdi_ref[1], 1.0))[:, 0:1]
    o_ref[...] = s * nd + b_ref[...]


def _row_spec(c):
    return pl.BlockSpec((BR, c), lambda i: (i, 0))


def _pair_spec(c):
    return pl.BlockSpec((2, BR, c), lambda i: (0, i, 0))


def _full_spec(r, c):
    return pl.BlockSpec((r, c), lambda i: (0, 0))


def _stage0(x, w, pdo):
    return pl.pallas_call(
        _stage0_body,
        grid=(N // BR,),
        in_specs=[_row_spec(F_IN), _full_spec(F_IN, F_HID), _pair_spec(16)],
        out_specs=_row_spec(F_HID),
        out_shape=jax.ShapeDtypeStruct((N, F_HID), jnp.float32),
    )(x, w, pdo)


def _stage_mid(p, pdi, pdo, b, w, c_in, c_out):
    return pl.pallas_call(
        _stage_mid_body,
        grid=(N // BR,),
        in_specs=[_pair_spec(c_in), _pair_spec(16), _pair_spec(16),
                  _full_spec(1, c_in), _full_spec(c_in, c_out)],
        out_specs=_row_spec(c_out),
        out_shape=jax.ShapeDtypeStruct((N, c_out), jnp.float32),
    )(p, pdi, pdo, b, w)


def _stage_fin(p, pdi, b, c):
    return pl.pallas_call(
        _stage_fin_body,
        grid=(N // BR,),
        in_specs=[_pair_spec(c), _pair_spec(16), _full_spec(1, c)],
        out_specs=_row_spec(c),
        out_shape=jax.ShapeDtypeStruct((N, c), jnp.float32),
    )(p, pdi, b)


def kernel(x, edge_index, W1, b1, W2, b2, W3, b3):
    src = edge_index[0].astype(jnp.int32)
    dst = edge_index[1].astype(jnp.int32)
    pad = E_PAD - E
    dummy = jnp.full((pad,), N, jnp.int32)
    src_deg = jnp.concatenate([src, dummy]).reshape(E_PAD // CHUNK, CHUNK)
    src_agg = jnp.concatenate([src, jnp.zeros((pad,), jnp.int32)]
                              ).reshape(E_PAD // CHUNK, CHUNK)
    dst_p = jnp.concatenate([dst, dummy]).reshape(E_PAD // CHUNK, CHUNK)

    ones16 = jnp.ones((CHUNK, 16), jnp.float32)
    z16 = jnp.zeros((ROWS_PER_TILE, 16), jnp.float32)
    z128 = jnp.zeros((ROWS_PER_TILE, 128), jnp.float32)
    z64 = jnp.zeros((ROWS_PER_TILE, 64), jnp.float32)

    pdo, pdi = _make_deg()(src_deg, dst_p, ones16, z16)
    pdo = pdo[:, :N, :]
    pdi = pdi[:, :N, :]

    agg128 = _make_agg(F_HID)
    agg64 = _make_agg(F_OUT)

    y1 = _stage0(x, W1, pdo)
    p1 = agg128(y1, src_agg, dst_p, z128)[:, :N, :]
    y2 = _stage_mid(p1, pdi, pdo, b1.reshape(1, -1), W2, F_HID, F_HID)
    p2 = agg128(y2, src_agg, dst_p, z128)[:, :N, :]
    y3 = _stage_mid(p2, pdi, pdo, b2.reshape(1, -1), W3, F_HID, F_OUT)
    p3 = agg64(y3, src_agg, dst_p, z64)[:, :N, :]
    return _stage_fin(p3, pdi, b3.reshape(1, -1), F_OUT)


# speedup vs baseline: 3.4082x; 3.4082x over previous
import functools

import jax
import jax.numpy as jnp
from jax import lax
from jax.experimental import pallas as pl
from jax.experimental.pallas import tpu as pltpu
from jax.experimental.pallas import tpu_sc as plsc

N = 10000
E = 320000
F_IN = 128
F_HID = 128
F_OUT = 64

NC = 2
NS = 16
NW = NC * NS
CHUNK = 128
CH_PER_W = 80
E_PAD = NW * CH_PER_W * CHUNK
N_ACC = 10240
ROWS_PER_TILE = N_ACC // NS
BR = 1000


def _mesh():
    return plsc.VectorSubcoreMesh(core_axis_name="c", subcore_axis_name="s")


def _deg_body(srcm, dstm, out_o, out_i,
              srcv, dstv, dego, degi, stage_o, stage_i, rbuf, obuf):
    cid = lax.axis_index("c")
    sid = lax.axis_index("s")
    wid = sid * NC + cid
    r0 = sid * ROWS_PER_TILE

    zv = jnp.zeros((16,), jnp.float32)
    onev = jnp.ones((16,), jnp.float32)

    def zbody(z, c):
        dego[pl.ds(z * 16, 16)] = zv
        degi[pl.ds(z * 16, 16)] = zv
        return c

    lax.fori_loop(0, N_ACC // 16, zbody, 0)
    e_per_w = CH_PER_W * CHUNK
    pltpu.sync_copy(srcm.at[pl.ds(wid * e_per_w, e_per_w)], srcv)
    pltpu.sync_copy(dstm.at[pl.ds(wid * e_per_w, e_per_w)], dstv)

    def sbody(g, c):
        si = srcv[pl.ds(g * 16, 16)]
        plsc.addupdate_scatter(dego, [si], onev)
        di = dstv[pl.ds(g * 16, 16)]
        plsc.addupdate_scatter(degi, [di], onev)
        return c

    lax.fori_loop(0, e_per_w // 16, sbody, 0)
    pltpu.sync_copy(dego, stage_o.at[sid])
    pltpu.sync_copy(degi, stage_i.at[sid])
    plsc.subcore_barrier()

    for stage, out in ((stage_o, out_o), (stage_i, out_i)):
        pltpu.sync_copy(stage.at[:, pl.ds(r0, ROWS_PER_TILE)], rbuf)

        def rbody(g, c):
            s = rbuf[0, pl.ds(g * 16, 16)]
            for r in range(1, NS):
                s = s + rbuf[r, pl.ds(g * 16, 16)]
            obuf[pl.ds(g * 16, 16)] = s
            return c

        lax.fori_loop(0, ROWS_PER_TILE // 16, rbody, 0)
        pltpu.sync_copy(obuf, out.at[cid, pl.ds(r0, ROWS_PER_TILE)])


def _make_deg():
    return functools.partial(
        pl.kernel,
        mesh=_mesh(),
        out_type=[
            jax.ShapeDtypeStruct((NC, N_ACC), jnp.float32),
            jax.ShapeDtypeStruct((NC, N_ACC), jnp.float32),
        ],
        scratch_types=[
            pltpu.VMEM((CH_PER_W * CHUNK,), jnp.int32),
            pltpu.VMEM((CH_PER_W * CHUNK,), jnp.int32),
            pltpu.VMEM((N_ACC,), jnp.float32),
            pltpu.VMEM((N_ACC,), jnp.float32),
            pltpu.VMEM_SHARED((NS, N_ACC), jnp.float32),
            pltpu.VMEM_SHARED((NS, N_ACC), jnp.float32),
            pltpu.VMEM((NS, ROWS_PER_TILE), jnp.float32),
            pltpu.VMEM((ROWS_PER_TILE,), jnp.float32),
        ],
        compiler_params=pltpu.CompilerParams(needs_layout_passes=False),
    )(_deg_body)


def _agg_body(y_hbm, srcm, dstm, zeros_hbm, out_hbm,
              srcv, dstv, rowb, acc, gsem):
    cid = lax.axis_index("c")
    sid = lax.axis_index("s")
    wid = sid * NC + cid
    r0 = sid * ROWS_PER_TILE
    pltpu.sync_copy(zeros_hbm, acc.at[pl.ds(r0, ROWS_PER_TILE)])
    pltpu.sync_copy(srcm.at[pl.ds(wid * CH_PER_W, CH_PER_W)], srcv)
    pltpu.sync_copy(dstm.at[pl.ds(wid * CH_PER_W, CH_PER_W)], dstv)
    plsc.subcore_barrier()

    def body(j, carry):
        pltpu.async_copy(y_hbm.at[srcv.at[j]], rowb, gsem).wait()
        pltpu.sync_copy(rowb, acc.at[dstv.at[j]], add=True)
        return carry

    lax.fori_loop(0, CH_PER_W, body, 0)
    plsc.subcore_barrier()
    pltpu.sync_copy(acc.at[pl.ds(r0, ROWS_PER_TILE)],
                    out_hbm.at[cid, pl.ds(r0, ROWS_PER_TILE)])


def _make_agg(C):
    return functools.partial(
        pl.kernel,
        mesh=_mesh(),
        out_type=jax.ShapeDtypeStruct((NC, N_ACC, C), jnp.float32),
        scratch_types=[
            pltpu.VMEM((CH_PER_W, CHUNK), jnp.int32),
            pltpu.VMEM((CH_PER_W, CHUNK), jnp.int32),
            pltpu.VMEM((CHUNK, C), jnp.float32),
            pltpu.VMEM_SHARED((N_ACC, C), jnp.float32),
            pltpu.SemaphoreType.DMA,
        ],
    )(_agg_body)


def _stage0_body(x_ref, w_ref, pdo_ref, o_ref):
    ns = lax.rsqrt(jnp.maximum(pdo_ref[0] + pdo_ref[1], 1.0))
    o_ref[...] = jnp.dot(x_ref[...] * ns, w_ref[...],
                         preferred_element_type=jnp.float32)


def _stage_mid_body(p_ref, pdi_ref, pdo_ref, b_ref, w_ref, o_ref):
    s = p_ref[0] + p_ref[1]
    nd = lax.rsqrt(jnp.maximum(pdi_ref[0] + pdi_ref[1], 1.0))
    h = jnp.maximum(s * nd + b_ref[...], 0.0)
    ns = lax.rsqrt(jnp.maximum(pdo_ref[0] + pdo_ref[1], 1.0))
    o_ref[...] = jnp.dot(h * ns, w_ref[...],
                         preferred_element_type=jnp.float32)


def _stage_fin_body(p_ref, pdi_ref, b_ref, o_ref):
    s = p_ref[0] + p_ref[1]
    nd = lax.rsqrt(jnp.maximum(pdi_ref[0] + pdi_ref[1], 1.0))
    o_ref[...] = s * nd + b_ref[...]


def _row_spec(c):
    return pl.BlockSpec((BR, c), lambda i: (i, 0))


def _pair_spec(c):
    return pl.BlockSpec((2, BR, c), lambda i: (0, i, 0))


def _deg_spec():
    return pl.BlockSpec((2, BR, 1), lambda i: (0, i, 0))


def _full_spec(r, c):
    return pl.BlockSpec((r, c), lambda i: (0, 0))


def _stage0(x, w, pdo):
    return pl.pallas_call(
        _stage0_body,
        grid=(N // BR,),
        in_specs=[_row_spec(F_IN), _full_spec(F_IN, F_HID), _deg_spec()],
        out_specs=_row_spec(F_HID),
        out_shape=jax.ShapeDtypeStruct((N, F_HID), jnp.float32),
    )(x, w, pdo)


def _stage_mid(p, pdi, pdo, b, w, c_in, c_out):
    return pl.pallas_call(
        _stage_mid_body,
        grid=(N // BR,),
        in_specs=[_pair_spec(c_in), _deg_spec(), _deg_spec(),
                  _full_spec(1, c_in), _full_spec(c_in, c_out)],
        out_specs=_row_spec(c_out),
        out_shape=jax.ShapeDtypeStruct((N, c_out), jnp.float32),
    )(p, pdi, pdo, b, w)


def _stage_fin(p, pdi, b, c):
    return pl.pallas_call(
        _stage_fin_body,
        grid=(N // BR,),
        in_specs=[_pair_spec(c), _deg_spec(), _full_spec(1, c)],
        out_specs=_row_spec(c),
        out_shape=jax.ShapeDtypeStruct((N, c), jnp.float32),
    )(p, pdi, b)


def kernel(x, edge_index, W1, b1, W2, b2, W3, b3):
    src = edge_index[0].astype(jnp.int32)
    dst = edge_index[1].astype(jnp.int32)
    pad = E_PAD - E
    dummy = jnp.full((pad,), N, jnp.int32)
    src_deg = jnp.concatenate([src, dummy])
    dst_deg = jnp.concatenate([dst, dummy])
    src_agg = jnp.concatenate([src, jnp.zeros((pad,), jnp.int32)]
                              ).reshape(E_PAD // CHUNK, CHUNK)
    dst_p = dst_deg.reshape(E_PAD // CHUNK, CHUNK)

    z128 = jnp.zeros((ROWS_PER_TILE, 128), jnp.float32)

    pdo, pdi = _make_deg()(src_deg, dst_deg)
    pdo = pdo[:, :N].reshape(NC, N, 1)
    pdi = pdi[:, :N].reshape(NC, N, 1)

    agg128 = _make_agg(F_HID)

    W3p = jnp.pad(W3, ((0, 0), (0, F_HID - F_OUT)))

    y1 = _stage0(x, W1, pdo)
    p1 = agg128(y1, src_agg, dst_p, z128)[:, :N, :]
    y2 = _stage_mid(p1, pdi, pdo, b1.reshape(1, -1), W2, F_HID, F_HID)
    p2 = agg128(y2, src_agg, dst_p, z128)[:, :N, :]
    y3 = _stage_mid(p2, pdi, pdo, b2.reshape(1, -1), W3p, F_HID, F_HID)
    p3 = agg128(y3, src_agg, dst_p, z128)[:, :N, :F_OUT]
    return _stage_fin(p3, pdi, b3.reshape(1, -1), F_OUT)
